# baseline (speedup 1.0000x reference)
import functools

import jax
import jax.numpy as jnp
from jax import lax
from jax.experimental import pallas as pl
from jax.experimental.pallas import tpu as pltpu
from jax.experimental.pallas import tpu_sc as plsc

N_NODES = 10000
N_EDGES = 320000
D = 128
BATCH = 4096

NC = 2
NS = 16
NW = NC * NS
CHUNK = 128
G = 8
NG = 10
NCH = G * NG
NCH_PAD = NCH + 2
E_PAD = NW * NCH * CHUNK
NP = 10240
ROWS_PER_TILE = NP // NS
TRASH = N_NODES


def _make_agg(with_deg: bool):
    mesh = plsc.VectorSubcoreMesh(core_axis_name="c", subcore_axis_name="s")
    out_type = [jax.ShapeDtypeStruct((NC, NP, D), jnp.float32)]
    scratch = [
        pltpu.VMEM((CHUNK,), jnp.int32),
        pltpu.VMEM((CHUNK,), jnp.int32),
        pltpu.VMEM((CHUNK,), jnp.int32),
        pltpu.VMEM((CHUNK,), jnp.int32),
        pltpu.VMEM((CHUNK, D), jnp.float32),
        pltpu.VMEM((CHUNK, D), jnp.float32),
        pltpu.VMEM_SHARED((NP, D), jnp.float32),
        pltpu.SemaphoreType.DMA,
        pltpu.SemaphoreType.DMA,
    ]
    if with_deg:
        out_type.append(jax.ShapeDtypeStruct((NC, NP), jnp.float32))
        scratch += [
            pltpu.VMEM((CHUNK,), jnp.float32),
            pltpu.VMEM_SHARED((NP,), jnp.float32),
        ]

    def body(x_hbm, src_hbm, dst_hbm, zeros_hbm, dzeros_hbm, ones_hbm,
             out_p, *rest):
        if with_deg:
            (out_deg, src_a, dst_a, src_b, dst_b, gbuf_a, gbuf_b, agg_sh,
             sem_a, sem_b, ones_v, deg_sh) = rest
        else:
            (src_a, dst_a, src_b, dst_b, gbuf_a, gbuf_b, agg_sh,
             sem_a, sem_b) = rest
        c = lax.axis_index("c")
        s = lax.axis_index("s")
        wid = c * NS + s
        pltpu.sync_copy(zeros_hbm, agg_sh.at[pl.ds(s * ROWS_PER_TILE, ROWS_PER_TILE)])
        if with_deg:
            pltpu.sync_copy(dzeros_hbm,
                            deg_sh.at[pl.ds(s * ROWS_PER_TILE, ROWS_PER_TILE)])
            pltpu.sync_copy(ones_hbm, ones_v)
        plsc.subcore_barrier()

        def stage(j, src_v, dst_v):
            pltpu.sync_copy(src_hbm.at[wid, j], src_v)
            pltpu.sync_copy(dst_hbm.at[wid, j], dst_v)

        def scatter(src_v, dst_v, gbuf, sem):
            pltpu.make_async_copy(x_hbm.at[src_v], gbuf, sem).wait()
            pltpu.sync_copy(gbuf, agg_sh.at[dst_v], add=True)
            if with_deg:
                pltpu.sync_copy(ones_v, deg_sh.at[dst_v], add=True)

        stage(0, src_a, dst_a)
        pltpu.async_copy(x_hbm.at[src_a], gbuf_a, sem_a)

        def pair_body(j2, carry):
            a = 2 * j2
            stage(a + 1, src_b, dst_b)
            pltpu.async_copy(x_hbm.at[src_b], gbuf_b, sem_b)
            scatter(src_a, dst_a, gbuf_a, sem_a)
            stage(a + 2, src_a, dst_a)
            pltpu.async_copy(x_hbm.at[src_a], gbuf_a, sem_a)
            scatter(src_b, dst_b, gbuf_b, sem_b)
            return carry

        lax.fori_loop(0, NCH // 2, pair_body, 0)
        pltpu.make_async_copy(x_hbm.at[src_a], gbuf_a, sem_a).wait()
        plsc.subcore_barrier()
        sl = pl.ds(s * ROWS_PER_TILE, ROWS_PER_TILE)
        pltpu.sync_copy(agg_sh.at[sl], out_p.at[c, sl])
        if with_deg:
            pltpu.sync_copy(deg_sh.at[sl], out_deg.at[c, sl])

    return pl.kernel(body, out_type=tuple(out_type) if with_deg else out_type[0],
                     mesh=mesh, scratch_types=tuple(scratch))


_agg_deg = _make_agg(True)
_agg = _make_agg(False)


def _tc_layer1_body(p_ref, degp_ref, x_ref, w_ref, b_ref, h_ref, degc_ref):
    agg = p_ref[0] + p_ref[1]
    deg = degp_ref[0] + degp_ref[1]
    degc = jnp.maximum(deg, 1.0)
    h = agg / degc[:, None]
    y = jnp.dot(h, w_ref[...], preferred_element_type=jnp.float32) + b_ref[...]
    h_ref[...] = jnp.maximum(y, 0.0) + x_ref[...]
    degc_ref[...] = degc


def _tc_layer2_body(p_ref, degc_ref, x_ref, w_ref, b_ref, h_ref):
    agg = p_ref[0] + p_ref[1]
    h = agg / degc_ref[...][:, None]
    y = jnp.dot(h, w_ref[...], preferred_element_type=jnp.float32) + b_ref[...]
    h_ref[...] = jnp.maximum(y, 0.0) + x_ref[...]


_R = 2048


def _tc_layer1(p, degp, x, w, b):
    grid = (NP // _R,)
    return pl.pallas_call(
        _tc_layer1_body,
        grid=grid,
        in_specs=[
            pl.BlockSpec((NC, _R, D), lambda i: (0, i, 0)),
            pl.BlockSpec((NC, _R), lambda i: (0, i)),
            pl.BlockSpec((_R, D), lambda i: (i, 0)),
            pl.BlockSpec((D, D), lambda i: (0, 0)),
            pl.BlockSpec((D,), lambda i: (0,)),
        ],
        out_specs=[
            pl.BlockSpec((_R, D), lambda i: (i, 0)),
            pl.BlockSpec((_R,), lambda i: (i,)),
        ],
        out_shape=[
            jax.ShapeDtypeStruct((NP, D), jnp.float32),
            jax.ShapeDtypeStruct((NP,), jnp.float32),
        ],
    )(p, degp, x, w, b)


def _tc_layer2(p, degc, x, w, b):
    grid = (NP // _R,)
    return pl.pallas_call(
        _tc_layer2_body,
        grid=grid,
        in_specs=[
            pl.BlockSpec((NC, _R, D), lambda i: (0, i, 0)),
            pl.BlockSpec((_R,), lambda i: (i,)),
            pl.BlockSpec((_R, D), lambda i: (i, 0)),
            pl.BlockSpec((D, D), lambda i: (0, 0)),
            pl.BlockSpec((D,), lambda i: (0,)),
        ],
        out_specs=pl.BlockSpec((_R, D), lambda i: (i, 0)),
        out_shape=jax.ShapeDtypeStruct((NP, D), jnp.float32),
    )(p, degc, x, w, b)


PAIRS_PER_TILE = BATCH // NW


def _head_body(h_hbm, i1_hbm, i2_hbm, out_hbm, i1_v, i2_v, b1_v, b2_v,
               ob2_v, sem):
    c = lax.axis_index("c")
    s = lax.axis_index("s")
    wid = c * NS + s
    pltpu.sync_copy(i1_hbm.at[wid], i1_v)
    pltpu.sync_copy(i2_hbm.at[wid], i2_v)
    pltpu.async_copy(h_hbm.at[i1_v], b1_v, sem).wait()
    pltpu.async_copy(h_hbm.at[i2_v], b2_v, sem).wait()

    def pbody(p, carry):
        for k in range(D // 16):
            v1 = b1_v[p, pl.ds(k * 16, 16)]
            v2 = b2_v[p, pl.ds(k * 16, 16)]
            ob2_v[p, pl.ds(k * 16, 16)] = v1 * v2
        return carry

    lax.fori_loop(0, PAIRS_PER_TILE, pbody, 0)
    pltpu.sync_copy(ob2_v, out_hbm.at[pl.ds(wid * PAIRS_PER_TILE, PAIRS_PER_TILE)])


_head = pl.kernel(
    _head_body,
    out_type=jax.ShapeDtypeStruct((BATCH, D), jnp.float32),
    mesh=plsc.VectorSubcoreMesh(core_axis_name="c", subcore_axis_name="s"),
    scratch_types=(
        pltpu.VMEM((PAIRS_PER_TILE,), jnp.int32),
        pltpu.VMEM((PAIRS_PER_TILE,), jnp.int32),
        pltpu.VMEM((PAIRS_PER_TILE, D), jnp.float32),
        pltpu.VMEM((PAIRS_PER_TILE, D), jnp.float32),
        pltpu.VMEM((PAIRS_PER_TILE, D), jnp.float32),
        pltpu.SemaphoreType.DMA,
    ),
)


def _tc_reduce_body(ps_ref, out_ref):
    out_ref[...] = jnp.sum(ps_ref[...], axis=-1)


def _tc_reduce(ps):
    return pl.pallas_call(
        _tc_reduce_body,
        out_shape=jax.ShapeDtypeStruct((BATCH,), jnp.float32),
    )(ps)


def kernel(inputs, node_feature, edge_index, W1, b1, W2, b2):
    ei = edge_index.astype(jnp.int32)
    pad = E_PAD - N_EDGES
    src3 = jnp.concatenate([ei[0], jnp.zeros((pad,), jnp.int32)]).reshape(NW, NCH, CHUNK)
    dst3 = jnp.concatenate([ei[1], jnp.full((pad,), TRASH, jnp.int32)]).reshape(NW, NCH, CHUNK)
    src3 = jnp.pad(src3, ((0, 0), (0, NCH_PAD - NCH), (0, 0)))
    dst3 = jnp.pad(dst3, ((0, 0), (0, NCH_PAD - NCH), (0, 0)), constant_values=TRASH)
    x0 = jnp.pad(node_feature, ((0, NP - N_NODES), (0, 0)))
    zeros_h = jnp.zeros((ROWS_PER_TILE, D), jnp.float32)
    dzeros_h = jnp.zeros((ROWS_PER_TILE,), jnp.float32)
    ones_h = jnp.ones((CHUNK,), jnp.float32)

    p1, degp = _agg_deg(x0, src3, dst3, zeros_h, dzeros_h, ones_h)
    h1, degc = _tc_layer1(p1, degp, x0, W1, b1)
    p2 = _agg(h1, src3, dst3, zeros_h, dzeros_h, ones_h)
    h2 = _tc_layer2(p2, degc, h1, W2, b2)

    i1 = inputs[:, 0].astype(jnp.int32).reshape(NW, PAIRS_PER_TILE)
    i2 = inputs[:, 1].astype(jnp.int32).reshape(NW, PAIRS_PER_TILE)
    ps = _head(h2, i1, i2)
    return _tc_reduce(ps)


# speedup vs baseline: 1.2873x; 1.2873x over previous
import jax
import jax.numpy as jnp
from jax import lax
from jax.experimental import pallas as pl
from jax.experimental.pallas import tpu as pltpu
from jax.experimental.pallas import tpu_sc as plsc

N_NODES = 10000
N_EDGES = 320000
D = 128
BATCH = 4096

NC = 2
NS = 16
NW = NC * NS
CHUNK = 128
TOT_CHUNKS = 2560
E_PAD = TOT_CHUNKS * CHUNK
N0 = 106
N1 = 54
CH0 = NS * N0
assert CH0 + NS * N1 == TOT_CHUNKS
NP = 10240
ROWS_PER_TILE = NP // NS
TRASH = N_NODES


def _make_agg(with_deg: bool):
    mesh = plsc.VectorSubcoreMesh(core_axis_name="c", subcore_axis_name="s")
    out_type = [jax.ShapeDtypeStruct((NC, NP, D), jnp.float32)]
    scratch = [
        pltpu.VMEM((CHUNK,), jnp.int32),
        pltpu.VMEM((CHUNK,), jnp.int32),
        pltpu.VMEM((CHUNK, D), jnp.float32),
        pltpu.VMEM_SHARED((NP, D), jnp.float32),
        pltpu.SemaphoreType.DMA,
    ]
    if with_deg:
        out_type.append(jax.ShapeDtypeStruct((NC, NP), jnp.float32))
        scratch += [
            pltpu.VMEM((CHUNK,), jnp.float32),
            pltpu.VMEM_SHARED((NP,), jnp.float32),
        ]

    def body(x_hbm, src_hbm, dst_hbm, zeros_hbm, dzeros_hbm, ones_hbm,
             out_p, *rest):
        if with_deg:
            out_deg, src_v, dst_v, gbuf, agg_sh, sem, ones_v, deg_sh = rest
        else:
            src_v, dst_v, gbuf, agg_sh, sem = rest
        c = lax.axis_index("c")
        s = lax.axis_index("s")
        sl = pl.ds(s * ROWS_PER_TILE, ROWS_PER_TILE)
        pltpu.sync_copy(zeros_hbm, agg_sh.at[sl])
        if with_deg:
            pltpu.sync_copy(dzeros_hbm, deg_sh.at[sl])
            pltpu.sync_copy(ones_hbm, ones_v)
        plsc.subcore_barrier()

        n_my = jnp.where(c == 0, N0, N1)
        base = jnp.where(c == 0, s * N0, CH0 + s * N1)

        def chunk_body(j, carry):
            ch = base + j
            pltpu.sync_copy(src_hbm.at[ch], src_v)
            pltpu.sync_copy(dst_hbm.at[ch], dst_v)
            pltpu.async_copy(x_hbm.at[src_v], gbuf, sem).wait()
            pltpu.sync_copy(gbuf, agg_sh.at[dst_v], add=True)
            if with_deg:
                pltpu.sync_copy(ones_v, deg_sh.at[dst_v], add=True)
            return carry

        lax.fori_loop(0, n_my, chunk_body, 0)
        plsc.subcore_barrier()
        pltpu.sync_copy(agg_sh.at[sl], out_p.at[c, sl])
        if with_deg:
            pltpu.sync_copy(deg_sh.at[sl], out_deg.at[c, sl])

    return pl.kernel(body, out_type=tuple(out_type) if with_deg else out_type[0],
                     mesh=mesh, scratch_types=tuple(scratch))


_agg_deg = _make_agg(True)
_agg = _make_agg(False)


def _tc_layer1_body(p_ref, degp_ref, x_ref, w_ref, b_ref, h_ref, degc_ref):
    agg = p_ref[0] + p_ref[1]
    deg = degp_ref[0] + degp_ref[1]
    degc = jnp.maximum(deg, 1.0)
    h = agg / degc[:, None]
    y = jnp.dot(h, w_ref[...], preferred_element_type=jnp.float32) + b_ref[...]
    h_ref[...] = jnp.maximum(y, 0.0) + x_ref[...]
    degc_ref[...] = degc


def _tc_layer2_body(p_ref, degc_ref, x_ref, w_ref, b_ref, h_ref):
    agg = p_ref[0] + p_ref[1]
    h = agg / degc_ref[...][:, None]
    y = jnp.dot(h, w_ref[...], preferred_element_type=jnp.float32) + b_ref[...]
    h_ref[...] = jnp.maximum(y, 0.0) + x_ref[...]


_R = 2048


def _tc_layer1(p, degp, x, w, b):
    return pl.pallas_call(
        _tc_layer1_body,
        grid=(NP // _R,),
        in_specs=[
            pl.BlockSpec((NC, _R, D), lambda i: (0, i, 0)),
            pl.BlockSpec((NC, _R), lambda i: (0, i)),
            pl.BlockSpec((_R, D), lambda i: (i, 0)),
            pl.BlockSpec((D, D), lambda i: (0, 0)),
            pl.BlockSpec((D,), lambda i: (0,)),
        ],
        out_specs=[
            pl.BlockSpec((_R, D), lambda i: (i, 0)),
            pl.BlockSpec((_R,), lambda i: (i,)),
        ],
        out_shape=[
            jax.ShapeDtypeStruct((NP, D), jnp.float32),
            jax.ShapeDtypeStruct((NP,), jnp.float32),
        ],
    )(p, degp, x, w, b)


def _tc_layer2(p, degc, x, w, b):
    return pl.pallas_call(
        _tc_layer2_body,
        grid=(NP // _R,),
        in_specs=[
            pl.BlockSpec((NC, _R, D), lambda i: (0, i, 0)),
            pl.BlockSpec((_R,), lambda i: (i,)),
            pl.BlockSpec((_R, D), lambda i: (i, 0)),
            pl.BlockSpec((D, D), lambda i: (0, 0)),
            pl.BlockSpec((D,), lambda i: (0,)),
        ],
        out_specs=pl.BlockSpec((_R, D), lambda i: (i, 0)),
        out_shape=jax.ShapeDtypeStruct((NP, D), jnp.float32),
    )(p, degc, x, w, b)


PAIRS_PER_TILE = BATCH // NW


def _head_body(h_hbm, i1_hbm, i2_hbm, out_hbm, i1_v, i2_v, b1_v, b2_v,
               ob2_v, sem):
    c = lax.axis_index("c")
    s = lax.axis_index("s")
    wid = c * NS + s
    pltpu.sync_copy(i1_hbm.at[wid], i1_v)
    pltpu.sync_copy(i2_hbm.at[wid], i2_v)
    pltpu.async_copy(h_hbm.at[i1_v], b1_v, sem).wait()
    pltpu.async_copy(h_hbm.at[i2_v], b2_v, sem).wait()

    def pbody(p, carry):
        for k in range(D // 16):
            v1 = b1_v[p, pl.ds(k * 16, 16)]
            v2 = b2_v[p, pl.ds(k * 16, 16)]
            ob2_v[p, pl.ds(k * 16, 16)] = v1 * v2
        return carry

    lax.fori_loop(0, PAIRS_PER_TILE, pbody, 0)
    pltpu.sync_copy(ob2_v, out_hbm.at[pl.ds(wid * PAIRS_PER_TILE, PAIRS_PER_TILE)])


_head = pl.kernel(
    _head_body,
    out_type=jax.ShapeDtypeStruct((BATCH, D), jnp.float32),
    mesh=plsc.VectorSubcoreMesh(core_axis_name="c", subcore_axis_name="s"),
    scratch_types=(
        pltpu.VMEM((PAIRS_PER_TILE,), jnp.int32),
        pltpu.VMEM((PAIRS_PER_TILE,), jnp.int32),
        pltpu.VMEM((PAIRS_PER_TILE, D), jnp.float32),
        pltpu.VMEM((PAIRS_PER_TILE, D), jnp.float32),
        pltpu.VMEM((PAIRS_PER_TILE, D), jnp.float32),
        pltpu.SemaphoreType.DMA,
    ),
)


def _tc_reduce_body(ps_ref, out_ref):
    out_ref[...] = jnp.sum(ps_ref[...], axis=-1)


def _tc_reduce(ps):
    return pl.pallas_call(
        _tc_reduce_body,
        out_shape=jax.ShapeDtypeStruct((BATCH,), jnp.float32),
    )(ps)


def kernel(inputs, node_feature, edge_index, W1, b1, W2, b2):
    ei = edge_index.astype(jnp.int32)
    pad = E_PAD - N_EDGES
    src2 = jnp.concatenate([ei[0], jnp.zeros((pad,), jnp.int32)]).reshape(TOT_CHUNKS, CHUNK)
    dst2 = jnp.concatenate([ei[1], jnp.full((pad,), TRASH, jnp.int32)]).reshape(TOT_CHUNKS, CHUNK)
    x0 = jnp.pad(node_feature, ((0, NP - N_NODES), (0, 0)))
    zeros_h = jnp.zeros((ROWS_PER_TILE, D), jnp.float32)
    dzeros_h = jnp.zeros((ROWS_PER_TILE,), jnp.float32)
    ones_h = jnp.ones((CHUNK,), jnp.float32)

    p1, degp = _agg_deg(x0, src2, dst2, zeros_h, dzeros_h, ones_h)
    h1, degc = _tc_layer1(p1, degp, x0, W1, b1)
    p2 = _agg(h1, src2, dst2, zeros_h, dzeros_h, ones_h)
    h2 = _tc_layer2(p2, degc, h1, W2, b2)

    i1 = inputs[:, 0].astype(jnp.int32).reshape(NW, PAIRS_PER_TILE)
    i2 = inputs[:, 1].astype(jnp.int32).reshape(NW, PAIRS_PER_TILE)
    ps = _head(h2, i1, i2)
    return _tc_reduce(ps)


# speedup vs baseline: 1.4220x; 1.1046x over previous
import jax
import jax.numpy as jnp
from jax import lax
from jax.experimental import pallas as pl
from jax.experimental.pallas import tpu as pltpu
from jax.experimental.pallas import tpu_sc as plsc

N_NODES = 10000
N_EDGES = 320000
D = 128
BATCH = 4096

NC = 2
NS = 16
NW = NC * NS
CHUNK = 128
G = 8
TOT_CHUNKS = 2560
TOT_GROUPS = TOT_CHUNKS // G
SD_PAD = TOT_GROUPS + 2
E_PAD = TOT_CHUNKS * CHUNK
NG0 = 12
NG1 = 8
GRP0 = NS * NG0
assert GRP0 + NS * NG1 == TOT_GROUPS
assert NG0 % 2 == 0 and NG1 % 2 == 0
NP = 10240
ROWS_PER_TILE = NP // NS
TRASH = N_NODES


def _make_agg(with_deg: bool):
    mesh = plsc.VectorSubcoreMesh(core_axis_name="c", subcore_axis_name="s")
    out_type = [jax.ShapeDtypeStruct((NC, NP, D), jnp.float32)]
    scratch = [
        pltpu.VMEM((2, G, CHUNK), jnp.int32),
        pltpu.VMEM((2, G, CHUNK), jnp.int32),
        pltpu.VMEM((CHUNK, D), jnp.float32),
        pltpu.VMEM((CHUNK, D), jnp.float32),
        pltpu.VMEM_SHARED((NP, D), jnp.float32),
        pltpu.SemaphoreType.DMA,
        pltpu.SemaphoreType.DMA,
        pltpu.SemaphoreType.DMA,
        pltpu.SemaphoreType.DMA,
    ]
    if with_deg:
        out_type.append(jax.ShapeDtypeStruct((NC, NP), jnp.float32))
        scratch += [
            pltpu.VMEM((CHUNK,), jnp.float32),
            pltpu.VMEM_SHARED((NP,), jnp.float32),
        ]

    def body(x_hbm, sd_hbm, zeros_hbm, dzeros_hbm, ones_hbm, out_p, *rest):
        if with_deg:
            (out_deg, sd_a, sd_b, g_a, g_b, agg_sh, isem_a, isem_b,
             gsem_a, gsem_b, ones_v, deg_sh) = rest
        else:
            (sd_a, sd_b, g_a, g_b, agg_sh, isem_a, isem_b,
             gsem_a, gsem_b) = rest
        gbufs = (g_a, g_b)
        gsems = (gsem_a, gsem_b)
        c = lax.axis_index("c")
        s = lax.axis_index("s")
        sl = pl.ds(s * ROWS_PER_TILE, ROWS_PER_TILE)
        pltpu.sync_copy(zeros_hbm, agg_sh.at[sl])
        if with_deg:
            pltpu.sync_copy(dzeros_hbm, deg_sh.at[sl])
            pltpu.sync_copy(ones_hbm, ones_v)
        plsc.subcore_barrier()

        ng_my = jnp.where(c == 0, NG0, NG1)
        base_g = jnp.where(c == 0, s * NG0, GRP0 + s * NG1)

        def do_group(g_rel, sd_cur, sd_nxt, isem_cur, isem_nxt):
            for cc in range(G):
                buf = gbufs[cc % 2]
                sem = gsems[cc % 2]
                pltpu.make_async_copy(x_hbm.at[sd_cur.at[0, cc]], buf, sem).wait()
                if cc + 1 < G:
                    pltpu.async_copy(x_hbm.at[sd_cur.at[0, cc + 1]],
                                     gbufs[(cc + 1) % 2], gsems[(cc + 1) % 2])
                else:
                    pltpu.make_async_copy(sd_hbm.at[base_g + g_rel + 1],
                                          sd_nxt, isem_nxt).wait()
                    pltpu.async_copy(x_hbm.at[sd_nxt.at[0, 0]],
                                     gbufs[0], gsems[0])
                pltpu.sync_copy(buf, agg_sh.at[sd_cur.at[1, cc]], add=True)
                if with_deg:
                    pltpu.sync_copy(ones_v, deg_sh.at[sd_cur.at[1, cc]], add=True)
            pltpu.async_copy(sd_hbm.at[base_g + g_rel + 2], sd_cur, isem_cur)

        pltpu.async_copy(sd_hbm.at[base_g], sd_a, isem_a)
        pltpu.async_copy(sd_hbm.at[base_g + 1], sd_b, isem_b)
        pltpu.make_async_copy(sd_hbm.at[base_g], sd_a, isem_a).wait()
        pltpu.async_copy(x_hbm.at[sd_a.at[0, 0]], gbufs[0], gsems[0])

        def pair_body(i, carry):
            do_group(2 * i, sd_a, sd_b, isem_a, isem_b)
            do_group(2 * i + 1, sd_b, sd_a, isem_b, isem_a)
            return carry

        lax.fori_loop(0, ng_my // 2, pair_body, 0)
        pltpu.make_async_copy(x_hbm.at[sd_a.at[0, 0]], gbufs[0], gsems[0]).wait()
        pltpu.make_async_copy(sd_hbm.at[base_g], sd_b, isem_b).wait()
        plsc.subcore_barrier()
        pltpu.sync_copy(agg_sh.at[sl], out_p.at[c, sl])
        if with_deg:
            pltpu.sync_copy(deg_sh.at[sl], out_deg.at[c, sl])

    return pl.kernel(body, out_type=tuple(out_type) if with_deg else out_type[0],
                     mesh=mesh, scratch_types=tuple(scratch))


_agg_deg = _make_agg(True)
_agg = _make_agg(False)


def _tc_layer1_body(p_ref, degp_ref, x_ref, w_ref, b_ref, h_ref, degc_ref):
    agg = p_ref[0] + p_ref[1]
    deg = degp_ref[0] + degp_ref[1]
    degc = jnp.maximum(deg, 1.0)
    h = agg / degc[:, None]
    y = jnp.dot(h, w_ref[...], preferred_element_type=jnp.float32) + b_ref[...]
    h_ref[...] = jnp.maximum(y, 0.0) + x_ref[...]
    degc_ref[...] = degc


def _tc_layer2_body(p_ref, degc_ref, x_ref, w_ref, b_ref, h_ref):
    agg = p_ref[0] + p_ref[1]
    h = agg / degc_ref[...][:, None]
    y = jnp.dot(h, w_ref[...], preferred_element_type=jnp.float32) + b_ref[...]
    h_ref[...] = jnp.maximum(y, 0.0) + x_ref[...]


_R = 2048


def _tc_layer1(p, degp, x, w, b):
    return pl.pallas_call(
        _tc_layer1_body,
        grid=(NP // _R,),
        in_specs=[
            pl.BlockSpec((NC, _R, D), lambda i: (0, i, 0)),
            pl.BlockSpec((NC, _R), lambda i: (0, i)),
            pl.BlockSpec((_R, D), lambda i: (i, 0)),
            pl.BlockSpec((D, D), lambda i: (0, 0)),
            pl.BlockSpec((D,), lambda i: (0,)),
        ],
        out_specs=[
            pl.BlockSpec((_R, D), lambda i: (i, 0)),
            pl.BlockSpec((_R,), lambda i: (i,)),
        ],
        out_shape=[
            jax.ShapeDtypeStruct((NP, D), jnp.float32),
            jax.ShapeDtypeStruct((NP,), jnp.float32),
        ],
    )(p, degp, x, w, b)


def _tc_layer2(p, degc, x, w, b):
    return pl.pallas_call(
        _tc_layer2_body,
        grid=(NP // _R,),
        in_specs=[
            pl.BlockSpec((NC, _R, D), lambda i: (0, i, 0)),
            pl.BlockSpec((_R,), lambda i: (i,)),
            pl.BlockSpec((_R, D), lambda i: (i, 0)),
            pl.BlockSpec((D, D), lambda i: (0, 0)),
            pl.BlockSpec((D,), lambda i: (0,)),
        ],
        out_specs=pl.BlockSpec((_R, D), lambda i: (i, 0)),
        out_shape=jax.ShapeDtypeStruct((NP, D), jnp.float32),
    )(p, degc, x, w, b)


PAIRS_PER_TILE = BATCH // NW


def _head_body(h_hbm, i1_hbm, i2_hbm, out_hbm, i1_v, i2_v, b1_v, b2_v,
               ob2_v, sem):
    c = lax.axis_index("c")
    s = lax.axis_index("s")
    wid = c * NS + s
    pltpu.sync_copy(i1_hbm.at[wid], i1_v)
    pltpu.sync_copy(i2_hbm.at[wid], i2_v)
    pltpu.async_copy(h_hbm.at[i1_v], b1_v, sem).wait()
    pltpu.async_copy(h_hbm.at[i2_v], b2_v, sem).wait()

    def pbody(p, carry):
        for k in range(D // 16):
            v1 = b1_v[p, pl.ds(k * 16, 16)]
            v2 = b2_v[p, pl.ds(k * 16, 16)]
            ob2_v[p, pl.ds(k * 16, 16)] = v1 * v2
        return carry

    lax.fori_loop(0, PAIRS_PER_TILE, pbody, 0)
    pltpu.sync_copy(ob2_v, out_hbm.at[pl.ds(wid * PAIRS_PER_TILE, PAIRS_PER_TILE)])


_head = pl.kernel(
    _head_body,
    out_type=jax.ShapeDtypeStruct((BATCH, D), jnp.float32),
    mesh=plsc.VectorSubcoreMesh(core_axis_name="c", subcore_axis_name="s"),
    scratch_types=(
        pltpu.VMEM((PAIRS_PER_TILE,), jnp.int32),
        pltpu.VMEM((PAIRS_PER_TILE,), jnp.int32),
        pltpu.VMEM((PAIRS_PER_TILE, D), jnp.float32),
        pltpu.VMEM((PAIRS_PER_TILE, D), jnp.float32),
        pltpu.VMEM((PAIRS_PER_TILE, D), jnp.float32),
        pltpu.SemaphoreType.DMA,
    ),
)


def _tc_reduce_body(ps_ref, out_ref):
    out_ref[...] = jnp.sum(ps_ref[...], axis=-1)


def _tc_reduce(ps):
    return pl.pallas_call(
        _tc_reduce_body,
        out_shape=jax.ShapeDtypeStruct((BATCH,), jnp.float32),
    )(ps)


def kernel(inputs, node_feature, edge_index, W1, b1, W2, b2):
    ei = edge_index.astype(jnp.int32)
    pad = E_PAD - N_EDGES
    src3 = jnp.concatenate([ei[0], jnp.zeros((pad,), jnp.int32)]).reshape(TOT_GROUPS, G, CHUNK)
    dst3 = jnp.concatenate([ei[1], jnp.full((pad,), TRASH, jnp.int32)]).reshape(TOT_GROUPS, G, CHUNK)
    sd = jnp.stack([src3, dst3], axis=1)
    sd = jnp.pad(sd, ((0, SD_PAD - TOT_GROUPS), (0, 0), (0, 0), (0, 0)))
    x0 = jnp.pad(node_feature, ((0, NP - N_NODES), (0, 0)))
    zeros_h = jnp.zeros((ROWS_PER_TILE, D), jnp.float32)
    dzeros_h = jnp.zeros((ROWS_PER_TILE,), jnp.float32)
    ones_h = jnp.ones((CHUNK,), jnp.float32)

    p1, degp = _agg_deg(x0, sd, zeros_h, dzeros_h, ones_h)
    h1, degc = _tc_layer1(p1, degp, x0, W1, b1)
    p2 = _agg(h1, sd, zeros_h, dzeros_h, ones_h)
    h2 = _tc_layer2(p2, degc, h1, W2, b2)

    i1 = inputs[:, 0].astype(jnp.int32).reshape(NW, PAIRS_PER_TILE)
    i2 = inputs[:, 1].astype(jnp.int32).reshape(NW, PAIRS_PER_TILE)
    ps = _head(h2, i1, i2)
    return _tc_reduce(ps)


# speedup vs baseline: 1.5094x; 1.0614x over previous
import jax
import jax.numpy as jnp
from jax import lax
from jax.experimental import pallas as pl
from jax.experimental.pallas import tpu as pltpu
from jax.experimental.pallas import tpu_sc as plsc

N_NODES = 10000
N_EDGES = 320000
D = 128
BATCH = 4096

NC = 2
NS = 16
NW = NC * NS
CHUNK = 128
G = 8
TOT_CHUNKS = 2560
TOT_GROUPS = TOT_CHUNKS // G
SD_PAD = TOT_GROUPS + 2
E_PAD = TOT_CHUNKS * CHUNK
NG0 = 16
NG1 = 4
GRP0 = NS * NG0
assert GRP0 + NS * NG1 == TOT_GROUPS
assert NG0 % 2 == 0 and NG1 % 2 == 0
NP = 10240
ROWS_PER_TILE = NP // NS
TRASH = N_NODES


def _make_agg(with_deg: bool):
    mesh = plsc.VectorSubcoreMesh(core_axis_name="c", subcore_axis_name="s")
    out_type = [jax.ShapeDtypeStruct((NC, NP, D), jnp.float32)]
    scratch = [
        pltpu.VMEM((2, G, CHUNK), jnp.int32),
        pltpu.VMEM((2, G, CHUNK), jnp.int32),
        pltpu.VMEM((CHUNK, D), jnp.float32),
        pltpu.VMEM((CHUNK, D), jnp.float32),
        pltpu.VMEM_SHARED((NP, D), jnp.float32),
        pltpu.SemaphoreType.DMA,
        pltpu.SemaphoreType.DMA,
        pltpu.SemaphoreType.DMA,
        pltpu.SemaphoreType.DMA,
    ]
    if with_deg:
        out_type.append(jax.ShapeDtypeStruct((NC, NP), jnp.float32))
        scratch += [
            pltpu.VMEM((CHUNK,), jnp.float32),
            pltpu.VMEM_SHARED((NP,), jnp.float32),
        ]

    def body(x_hbm, sd_hbm, zeros_hbm, dzeros_hbm, ones_hbm, out_p, *rest):
        if with_deg:
            (out_deg, sd_a, sd_b, g_a, g_b, agg_sh, isem_a, isem_b,
             gsem_a, gsem_b, ones_v, deg_sh) = rest
        else:
            (sd_a, sd_b, g_a, g_b, agg_sh, isem_a, isem_b,
             gsem_a, gsem_b) = rest
        gbufs = (g_a, g_b)
        gsems = (gsem_a, gsem_b)
        c = lax.axis_index("c")
        s = lax.axis_index("s")
        sl = pl.ds(s * ROWS_PER_TILE, ROWS_PER_TILE)
        pltpu.sync_copy(zeros_hbm, agg_sh.at[sl])
        if with_deg:
            pltpu.sync_copy(dzeros_hbm, deg_sh.at[sl])
            pltpu.sync_copy(ones_hbm, ones_v)
        plsc.subcore_barrier()

        ng_my = jnp.where(c == 0, NG0, NG1)
        base_g = jnp.where(c == 0, s * NG0, GRP0 + s * NG1)

        def do_group(g_rel, sd_cur, sd_nxt, isem_cur, isem_nxt):
            for cc in range(G):
                buf = gbufs[cc % 2]
                sem = gsems[cc % 2]
                pltpu.make_async_copy(x_hbm.at[sd_cur.at[0, cc]], buf, sem).wait()
                if cc + 1 < G:
                    pltpu.async_copy(x_hbm.at[sd_cur.at[0, cc + 1]],
                                     gbufs[(cc + 1) % 2], gsems[(cc + 1) % 2])
                else:
                    pltpu.make_async_copy(sd_hbm.at[base_g + g_rel + 1],
                                          sd_nxt, isem_nxt).wait()
                    pltpu.async_copy(x_hbm.at[sd_nxt.at[0, 0]],
                                     gbufs[0], gsems[0])
                pltpu.sync_copy(buf, agg_sh.at[sd_cur.at[1, cc]], add=True)
                if with_deg:
                    pltpu.sync_copy(ones_v, deg_sh.at[sd_cur.at[1, cc]], add=True)
            pltpu.async_copy(sd_hbm.at[base_g + g_rel + 2], sd_cur, isem_cur)

        pltpu.async_copy(sd_hbm.at[base_g], sd_a, isem_a)
        pltpu.async_copy(sd_hbm.at[base_g + 1], sd_b, isem_b)
        pltpu.make_async_copy(sd_hbm.at[base_g], sd_a, isem_a).wait()
        pltpu.async_copy(x_hbm.at[sd_a.at[0, 0]], gbufs[0], gsems[0])

        def pair_body(i, carry):
            do_group(2 * i, sd_a, sd_b, isem_a, isem_b)
            do_group(2 * i + 1, sd_b, sd_a, isem_b, isem_a)
            return carry

        lax.fori_loop(0, ng_my // 2, pair_body, 0)
        pltpu.make_async_copy(x_hbm.at[sd_a.at[0, 0]], gbufs[0], gsems[0]).wait()
        pltpu.make_async_copy(sd_hbm.at[base_g], sd_b, isem_b).wait()
        plsc.subcore_barrier()
        pltpu.sync_copy(agg_sh.at[sl], out_p.at[c, sl])
        if with_deg:
            pltpu.sync_copy(deg_sh.at[sl], out_deg.at[c, sl])

    return pl.kernel(body, out_type=tuple(out_type) if with_deg else out_type[0],
                     mesh=mesh, scratch_types=tuple(scratch))


_agg_deg = _make_agg(True)
_agg = _make_agg(False)


def _tc_layer1_body(p_ref, degp_ref, x_ref, w_ref, b_ref, h_ref, degc_ref):
    agg = p_ref[0] + p_ref[1]
    deg = degp_ref[0] + degp_ref[1]
    degc = jnp.maximum(deg, 1.0)
    h = agg / degc[:, None]
    y = jnp.dot(h, w_ref[...], preferred_element_type=jnp.float32) + b_ref[...]
    h_ref[...] = jnp.maximum(y, 0.0) + x_ref[...]
    degc_ref[...] = degc


def _tc_layer2_body(p_ref, degc_ref, x_ref, w_ref, b_ref, h_ref):
    agg = p_ref[0] + p_ref[1]
    h = agg / degc_ref[...][:, None]
    y = jnp.dot(h, w_ref[...], preferred_element_type=jnp.float32) + b_ref[...]
    h_ref[...] = jnp.maximum(y, 0.0) + x_ref[...]


_R = 2048


def _tc_layer1(p, degp, x, w, b):
    return pl.pallas_call(
        _tc_layer1_body,
        grid=(NP // _R,),
        in_specs=[
            pl.BlockSpec((NC, _R, D), lambda i: (0, i, 0)),
            pl.BlockSpec((NC, _R), lambda i: (0, i)),
            pl.BlockSpec((_R, D), lambda i: (i, 0)),
            pl.BlockSpec((D, D), lambda i: (0, 0)),
            pl.BlockSpec((D,), lambda i: (0,)),
        ],
        out_specs=[
            pl.BlockSpec((_R, D), lambda i: (i, 0)),
            pl.BlockSpec((_R,), lambda i: (i,)),
        ],
        out_shape=[
            jax.ShapeDtypeStruct((NP, D), jnp.float32),
            jax.ShapeDtypeStruct((NP,), jnp.float32),
        ],
    )(p, degp, x, w, b)


def _tc_layer2(p, degc, x, w, b):
    return pl.pallas_call(
        _tc_layer2_body,
        grid=(NP // _R,),
        in_specs=[
            pl.BlockSpec((NC, _R, D), lambda i: (0, i, 0)),
            pl.BlockSpec((_R,), lambda i: (i,)),
            pl.BlockSpec((_R, D), lambda i: (i, 0)),
            pl.BlockSpec((D, D), lambda i: (0, 0)),
            pl.BlockSpec((D,), lambda i: (0,)),
        ],
        out_specs=pl.BlockSpec((_R, D), lambda i: (i, 0)),
        out_shape=jax.ShapeDtypeStruct((NP, D), jnp.float32),
    )(p, degc, x, w, b)


PAIRS_PER_TILE = BATCH // NW


def _head_body(h_hbm, i1_hbm, i2_hbm, out_hbm, i1_v, i2_v, b1_v, b2_v,
               ob2_v, sem):
    c = lax.axis_index("c")
    s = lax.axis_index("s")
    wid = c * NS + s
    pltpu.sync_copy(i1_hbm.at[wid], i1_v)
    pltpu.sync_copy(i2_hbm.at[wid], i2_v)
    pltpu.async_copy(h_hbm.at[i1_v], b1_v, sem).wait()
    pltpu.async_copy(h_hbm.at[i2_v], b2_v, sem).wait()

    def pbody(p, carry):
        for k in range(D // 16):
            v1 = b1_v[p, pl.ds(k * 16, 16)]
            v2 = b2_v[p, pl.ds(k * 16, 16)]
            ob2_v[p, pl.ds(k * 16, 16)] = v1 * v2
        return carry

    lax.fori_loop(0, PAIRS_PER_TILE, pbody, 0)
    pltpu.sync_copy(ob2_v, out_hbm.at[pl.ds(wid * PAIRS_PER_TILE, PAIRS_PER_TILE)])


_head = pl.kernel(
    _head_body,
    out_type=jax.ShapeDtypeStruct((BATCH, D), jnp.float32),
    mesh=plsc.VectorSubcoreMesh(core_axis_name="c", subcore_axis_name="s"),
    scratch_types=(
        pltpu.VMEM((PAIRS_PER_TILE,), jnp.int32),
        pltpu.VMEM((PAIRS_PER_TILE,), jnp.int32),
        pltpu.VMEM((PAIRS_PER_TILE, D), jnp.float32),
        pltpu.VMEM((PAIRS_PER_TILE, D), jnp.float32),
        pltpu.VMEM((PAIRS_PER_TILE, D), jnp.float32),
        pltpu.SemaphoreType.DMA,
    ),
)


def _tc_reduce_body(ps_ref, out_ref):
    out_ref[...] = jnp.sum(ps_ref[...], axis=-1)


def _tc_reduce(ps):
    return pl.pallas_call(
        _tc_reduce_body,
        out_shape=jax.ShapeDtypeStruct((BATCH,), jnp.float32),
    )(ps)


def kernel(inputs, node_feature, edge_index, W1, b1, W2, b2):
    ei = edge_index.astype(jnp.int32)
    pad = E_PAD - N_EDGES
    src3 = jnp.concatenate([ei[0], jnp.zeros((pad,), jnp.int32)]).reshape(TOT_GROUPS, G, CHUNK)
    dst3 = jnp.concatenate([ei[1], jnp.full((pad,), TRASH, jnp.int32)]).reshape(TOT_GROUPS, G, CHUNK)
    sd = jnp.stack([src3, dst3], axis=1)
    sd = jnp.pad(sd, ((0, SD_PAD - TOT_GROUPS), (0, 0), (0, 0), (0, 0)))
    x0 = jnp.pad(node_feature, ((0, NP - N_NODES), (0, 0)))
    zeros_h = jnp.zeros((ROWS_PER_TILE, D), jnp.float32)
    dzeros_h = jnp.zeros((ROWS_PER_TILE,), jnp.float32)
    ones_h = jnp.ones((CHUNK,), jnp.float32)

    p1, degp = _agg_deg(x0, sd, zeros_h, dzeros_h, ones_h)
    h1, degc = _tc_layer1(p1, degp, x0, W1, b1)
    p2 = _agg(h1, sd, zeros_h, dzeros_h, ones_h)
    h2 = _tc_layer2(p2, degc, h1, W2, b2)

    i1 = inputs[:, 0].astype(jnp.int32).reshape(NW, PAIRS_PER_TILE)
    i2 = inputs[:, 1].astype(jnp.int32).reshape(NW, PAIRS_PER_TILE)
    ps = _head(h2, i1, i2)
    return _tc_reduce(ps)


# speedup vs baseline: 1.5624x; 1.0351x over previous
import jax
import jax.numpy as jnp
from jax import lax
from jax.experimental import pallas as pl
from jax.experimental.pallas import tpu as pltpu
from jax.experimental.pallas import tpu_sc as plsc

N_NODES = 10000
N_EDGES = 320000
D = 128
BATCH = 4096

NC = 2
NS = 16
NW = NC * NS
CHUNK = 128
G = 8
TOT_CHUNKS = 2560
TOT_GROUPS = TOT_CHUNKS // G
SD_PAD = TOT_GROUPS + 2
E_PAD = TOT_CHUNKS * CHUNK
NG0 = 18
NG1 = 2
GRP0 = NS * NG0
assert GRP0 + NS * NG1 == TOT_GROUPS
assert NG0 % 2 == 0 and NG1 % 2 == 0
NP = 10240
ROWS_PER_TILE = NP // NS
TRASH = N_NODES


def _make_agg(with_deg: bool):
    mesh = plsc.VectorSubcoreMesh(core_axis_name="c", subcore_axis_name="s")
    out_type = [jax.ShapeDtypeStruct((NC, NP, D), jnp.float32)]
    scratch = [
        pltpu.VMEM((2, G, CHUNK), jnp.int32),
        pltpu.VMEM((2, G, CHUNK), jnp.int32),
        pltpu.VMEM((CHUNK, D), jnp.float32),
        pltpu.VMEM((CHUNK, D), jnp.float32),
        pltpu.VMEM_SHARED((NP, D), jnp.float32),
        pltpu.SemaphoreType.DMA,
        pltpu.SemaphoreType.DMA,
        pltpu.SemaphoreType.DMA,
        pltpu.SemaphoreType.DMA,
    ]
    if with_deg:
        out_type.append(jax.ShapeDtypeStruct((NC, NP), jnp.float32))
        scratch += [
            pltpu.VMEM((CHUNK,), jnp.float32),
            pltpu.VMEM_SHARED((NP,), jnp.float32),
        ]

    def body(x_hbm, sd_hbm, zeros_hbm, dzeros_hbm, ones_hbm, out_p, *rest):
        if with_deg:
            (out_deg, sd_a, sd_b, g_a, g_b, agg_sh, isem_a, isem_b,
             gsem_a, gsem_b, ones_v, deg_sh) = rest
        else:
            (sd_a, sd_b, g_a, g_b, agg_sh, isem_a, isem_b,
             gsem_a, gsem_b) = rest
        gbufs = (g_a, g_b)
        gsems = (gsem_a, gsem_b)
        c = lax.axis_index("c")
        s = lax.axis_index("s")
        sl = pl.ds(s * ROWS_PER_TILE, ROWS_PER_TILE)
        pltpu.sync_copy(zeros_hbm, agg_sh.at[sl])
        if with_deg:
            pltpu.sync_copy(dzeros_hbm, deg_sh.at[sl])
            pltpu.sync_copy(ones_hbm, ones_v)
        plsc.subcore_barrier()

        ng_my = jnp.where(c == 0, NG0, NG1)
        base_g = jnp.where(c == 0, s * NG0, GRP0 + s * NG1)

        def do_group(g_rel, sd_cur, sd_nxt, isem_cur, isem_nxt):
            for cc in range(G):
                buf = gbufs[cc % 2]
                sem = gsems[cc % 2]
                pltpu.make_async_copy(x_hbm.at[sd_cur.at[0, cc]], buf, sem).wait()
                if cc + 1 < G:
                    pltpu.async_copy(x_hbm.at[sd_cur.at[0, cc + 1]],
                                     gbufs[(cc + 1) % 2], gsems[(cc + 1) % 2])
                else:
                    pltpu.make_async_copy(sd_hbm.at[base_g + g_rel + 1],
                                          sd_nxt, isem_nxt).wait()
                    pltpu.async_copy(x_hbm.at[sd_nxt.at[0, 0]],
                                     gbufs[0], gsems[0])
                pltpu.sync_copy(buf, agg_sh.at[sd_cur.at[1, cc]], add=True)
                if with_deg:
                    pltpu.sync_copy(ones_v, deg_sh.at[sd_cur.at[1, cc]], add=True)
            pltpu.async_copy(sd_hbm.at[base_g + g_rel + 2], sd_cur, isem_cur)

        pltpu.async_copy(sd_hbm.at[base_g], sd_a, isem_a)
        pltpu.async_copy(sd_hbm.at[base_g + 1], sd_b, isem_b)
        pltpu.make_async_copy(sd_hbm.at[base_g], sd_a, isem_a).wait()
        pltpu.async_copy(x_hbm.at[sd_a.at[0, 0]], gbufs[0], gsems[0])

        def pair_body(i, carry):
            do_group(2 * i, sd_a, sd_b, isem_a, isem_b)
            do_group(2 * i + 1, sd_b, sd_a, isem_b, isem_a)
            return carry

        lax.fori_loop(0, ng_my // 2, pair_body, 0)
        pltpu.make_async_copy(x_hbm.at[sd_a.at[0, 0]], gbufs[0], gsems[0]).wait()
        pltpu.make_async_copy(sd_hbm.at[base_g], sd_b, isem_b).wait()
        plsc.subcore_barrier()
        pltpu.sync_copy(agg_sh.at[sl], out_p.at[c, sl])
        if with_deg:
            pltpu.sync_copy(deg_sh.at[sl], out_deg.at[c, sl])

    return pl.kernel(body, out_type=tuple(out_type) if with_deg else out_type[0],
                     mesh=mesh, scratch_types=tuple(scratch))


_agg_deg = _make_agg(True)
_agg = _make_agg(False)


def _tc_layer1_body(p_ref, degp_ref, x_ref, w_ref, b_ref, h_ref, degc_ref):
    agg = p_ref[0] + p_ref[1]
    deg = degp_ref[0] + degp_ref[1]
    degc = jnp.maximum(deg, 1.0)
    h = agg / degc[:, None]
    y = jnp.dot(h, w_ref[...], preferred_element_type=jnp.float32) + b_ref[...]
    h_ref[...] = jnp.maximum(y, 0.0) + x_ref[...]
    degc_ref[...] = degc


def _tc_layer2_body(p_ref, degc_ref, x_ref, w_ref, b_ref, h_ref):
    agg = p_ref[0] + p_ref[1]
    h = agg / degc_ref[...][:, None]
    y = jnp.dot(h, w_ref[...], preferred_element_type=jnp.float32) + b_ref[...]
    h_ref[...] = jnp.maximum(y, 0.0) + x_ref[...]


_R = 2048


def _tc_layer1(p, degp, x, w, b):
    return pl.pallas_call(
        _tc_layer1_body,
        grid=(NP // _R,),
        in_specs=[
            pl.BlockSpec((NC, _R, D), lambda i: (0, i, 0)),
            pl.BlockSpec((NC, _R), lambda i: (0, i)),
            pl.BlockSpec((_R, D), lambda i: (i, 0)),
            pl.BlockSpec((D, D), lambda i: (0, 0)),
            pl.BlockSpec((D,), lambda i: (0,)),
        ],
        out_specs=[
            pl.BlockSpec((_R, D), lambda i: (i, 0)),
            pl.BlockSpec((_R,), lambda i: (i,)),
        ],
        out_shape=[
            jax.ShapeDtypeStruct((NP, D), jnp.float32),
            jax.ShapeDtypeStruct((NP,), jnp.float32),
        ],
    )(p, degp, x, w, b)


def _tc_layer2(p, degc, x, w, b):
    return pl.pallas_call(
        _tc_layer2_body,
        grid=(NP // _R,),
        in_specs=[
            pl.BlockSpec((NC, _R, D), lambda i: (0, i, 0)),
            pl.BlockSpec((_R,), lambda i: (i,)),
            pl.BlockSpec((_R, D), lambda i: (i, 0)),
            pl.BlockSpec((D, D), lambda i: (0, 0)),
            pl.BlockSpec((D,), lambda i: (0,)),
        ],
        out_specs=pl.BlockSpec((_R, D), lambda i: (i, 0)),
        out_shape=jax.ShapeDtypeStruct((NP, D), jnp.float32),
    )(p, degc, x, w, b)


PAIRS_PER_TILE = BATCH // NW


def _head_body(h_hbm, i1_hbm, i2_hbm, out_hbm, i1_v, i2_v, b1_v, b2_v,
               ob2_v, sem):
    c = lax.axis_index("c")
    s = lax.axis_index("s")
    wid = c * NS + s
    pltpu.sync_copy(i1_hbm.at[wid], i1_v)
    pltpu.sync_copy(i2_hbm.at[wid], i2_v)
    pltpu.async_copy(h_hbm.at[i1_v], b1_v, sem).wait()
    pltpu.async_copy(h_hbm.at[i2_v], b2_v, sem).wait()

    def pbody(p, carry):
        for k in range(D // 16):
            v1 = b1_v[p, pl.ds(k * 16, 16)]
            v2 = b2_v[p, pl.ds(k * 16, 16)]
            ob2_v[p, pl.ds(k * 16, 16)] = v1 * v2
        return carry

    lax.fori_loop(0, PAIRS_PER_TILE, pbody, 0)
    pltpu.sync_copy(ob2_v, out_hbm.at[pl.ds(wid * PAIRS_PER_TILE, PAIRS_PER_TILE)])


_head = pl.kernel(
    _head_body,
    out_type=jax.ShapeDtypeStruct((BATCH, D), jnp.float32),
    mesh=plsc.VectorSubcoreMesh(core_axis_name="c", subcore_axis_name="s"),
    scratch_types=(
        pltpu.VMEM((PAIRS_PER_TILE,), jnp.int32),
        pltpu.VMEM((PAIRS_PER_TILE,), jnp.int32),
        pltpu.VMEM((PAIRS_PER_TILE, D), jnp.float32),
        pltpu.VMEM((PAIRS_PER_TILE, D), jnp.float32),
        pltpu.VMEM((PAIRS_PER_TILE, D), jnp.float32),
        pltpu.SemaphoreType.DMA,
    ),
)


def _tc_reduce_body(ps_ref, out_ref):
    out_ref[...] = jnp.sum(ps_ref[...], axis=-1)


def _tc_reduce(ps):
    return pl.pallas_call(
        _tc_reduce_body,
        out_shape=jax.ShapeDtypeStruct((BATCH,), jnp.float32),
    )(ps)


def kernel(inputs, node_feature, edge_index, W1, b1, W2, b2):
    ei = edge_index.astype(jnp.int32)
    pad = E_PAD - N_EDGES
    src3 = jnp.concatenate([ei[0], jnp.zeros((pad,), jnp.int32)]).reshape(TOT_GROUPS, G, CHUNK)
    dst3 = jnp.concatenate([ei[1], jnp.full((pad,), TRASH, jnp.int32)]).reshape(TOT_GROUPS, G, CHUNK)
    sd = jnp.stack([src3, dst3], axis=1)
    sd = jnp.pad(sd, ((0, SD_PAD - TOT_GROUPS), (0, 0), (0, 0), (0, 0)))
    x0 = jnp.pad(node_feature, ((0, NP - N_NODES), (0, 0)))
    zeros_h = jnp.zeros((ROWS_PER_TILE, D), jnp.float32)
    dzeros_h = jnp.zeros((ROWS_PER_TILE,), jnp.float32)
    ones_h = jnp.ones((CHUNK,), jnp.float32)

    p1, degp = _agg_deg(x0, sd, zeros_h, dzeros_h, ones_h)
    h1, degc = _tc_layer1(p1, degp, x0, W1, b1)
    p2 = _agg(h1, sd, zeros_h, dzeros_h, ones_h)
    h2 = _tc_layer2(p2, degc, h1, W2, b2)

    i1 = inputs[:, 0].astype(jnp.int32).reshape(NW, PAIRS_PER_TILE)
    i2 = inputs[:, 1].astype(jnp.int32).reshape(NW, PAIRS_PER_TILE)
    ps = _head(h2, i1, i2)
    return _tc_reduce(ps)
